# Initial kernel scaffold; baseline (speedup 1.0000x reference)
import jax
import jax.numpy as jnp
from jax.experimental import pallas as pl


def kernel(inputs, embed):
    raise NotImplementedError("write your pallas kernel here")


# speedup vs baseline: 1.9386x; 1.9386x over previous
import jax
import jax.numpy as jnp
from jax.experimental import pallas as pl
from jax.experimental.pallas import tpu as pltpu

EMB = 256
CODES = 1024
PIX = 1024
BATCH = 16
COMMIT = 0.25


def _vq_kernel(x_ref, e_ref, out_ref, oh_ref, loss_ref):
    n = pl.program_id(0)
    x = x_ref[0]
    e = e_ref[...]
    scores = jax.lax.dot_general(
        x, e, (((0,), (0,)), ((), ())), preferred_element_type=jnp.float32
    )
    e_norm = jnp.sum(e * e, axis=0, keepdims=True)
    d = e_norm - 2.0 * scores
    dmin = jnp.min(d, axis=1, keepdims=True)
    iota = jax.lax.broadcasted_iota(jnp.int32, (PIX, CODES), 1)
    idx = jnp.min(jnp.where(d == dmin, iota, CODES), axis=1, keepdims=True)
    oh = (iota == idx).astype(jnp.float32)
    oh_ref[...] = oh
    q = jax.lax.dot_general(
        e, oh, (((1,), (1,)), ((), ())), preferred_element_type=jnp.float32
    )
    out_ref[0] = q
    z_norm = jnp.sum(x * x, axis=0, keepdims=True)
    part = jnp.sum(z_norm) + jnp.sum(dmin)

    @pl.when(n == 0)
    def _():
        loss_ref[...] = jnp.zeros_like(loss_ref)

    loss_ref[...] += part


def kernel(inputs, embed):
    x = inputs.reshape(BATCH, EMB, PIX)
    out, onehot, loss_sum = pl.pallas_call(
        _vq_kernel,
        grid=(BATCH,),
        in_specs=[
            pl.BlockSpec((1, EMB, PIX), lambda n: (n, 0, 0)),
            pl.BlockSpec((EMB, CODES), lambda n: (0, 0)),
        ],
        out_specs=[
            pl.BlockSpec((1, EMB, PIX), lambda n: (n, 0, 0)),
            pl.BlockSpec((PIX, CODES), lambda n: (n, 0)),
            pl.BlockSpec((1, 1), lambda n: (0, 0)),
        ],
        out_shape=[
            jax.ShapeDtypeStruct((BATCH, EMB, PIX), jnp.float32),
            jax.ShapeDtypeStruct((BATCH * PIX, CODES), jnp.float32),
            jax.ShapeDtypeStruct((1, 1), jnp.float32),
        ],
        compiler_params=pltpu.CompilerParams(
            dimension_semantics=("arbitrary",),
        ),
    )(x, embed)
    loss = loss_sum[0, 0] * ((1.0 + COMMIT) / (BATCH * PIX * EMB))
    return (out.reshape(BATCH, EMB, 32, 32), loss, onehot)
